# baseline (speedup 1.0000x reference)
import jax
import jax.numpy as jnp
from jax import lax
from jax.experimental import pallas as pl
from jax.experimental.pallas import tpu as pltpu

_BLK = 128
_KB = 18


def _gcn_body(adj0_ref, adj1_ref, x_ref, aw_ref, ab_ref, w1_ref, b1_ref,
              wm_ref, bm_ref, w2_ref, b2_ref, out_ref,
              adj_vmem, h_ref, s_ref, s3_ref,
              z1_ref, z2_ref, nz0_ref, nz1_ref):
    p = pl.program_id(0)
    i = pl.program_id(1)

    @pl.when(p == 0)
    def _phase0():
        a0 = adj0_ref[...]
        a1 = adj1_ref[...]
        w = aw_ref[...]
        dn = (((1,), (1,)), ((), ()))
        z1_ref[:, pl.ds(i * _BLK, _BLK)] = lax.dot_general(
            w, a0, dn, preferred_element_type=jnp.float32)
        z2_ref[:, pl.ds(i * _BLK, _BLK)] = lax.dot_general(
            w, a1, dn, preferred_element_type=jnp.float32)

        @pl.when(i < _KB)
        def _():
            adj_vmem[pl.ds(i * _BLK, _BLK), :] = a0

    @pl.when(p == 1)
    def _phase1():
        @pl.when(i == 0)
        def _():
            z1 = z1_ref[...] + ab_ref[...]
            z2 = z2_ref[...] + ab_ref[...]
            m = jnp.maximum(z1, z2)
            e1 = jnp.exp(z1 - m)
            e2 = jnp.exp(z2 - m)
            den = e1 + e2
            nz0_ref[...] = e1 / den
            nz1_ref[...] = e2 / den
            s_ref[...] = jnp.dot(x_ref[...], w1_ref[...],
                                  preferred_element_type=jnp.float32)

        nz0 = nz0_ref[...]
        nz1 = nz1_ref[...]
        a1 = adj1_ref[...]
        s1 = s_ref[...]
        b1 = b1_ref[...]

        @pl.when(i < _KB)
        def _():
            am = nz0 * adj_vmem[pl.ds(i * _BLK, _BLK), :] + nz1 * a1
            adj_vmem[pl.ds(i * _BLK, _BLK), :] = am
            h_ref[pl.ds(i * _BLK, _BLK), :] = jnp.maximum(
                jnp.dot(am, s1, preferred_element_type=jnp.float32) + b1, 0.0)

        @pl.when(i >= _KB)
        def _():
            am = nz0 * adj0_ref[...] + nz1 * a1
            h_ref[pl.ds(i * _BLK, _BLK), :] = jnp.maximum(
                jnp.dot(am, s1, preferred_element_type=jnp.float32) + b1, 0.0)

    @pl.when(p == 2)
    def _phase2():
        @pl.when(i == 0)
        def _():
            s_ref[...] = jnp.dot(h_ref[...], wm_ref[...],
                                  preferred_element_type=jnp.float32)

        s2 = s_ref[...]
        bm = bm_ref[...]

        @pl.when(i < _KB)
        def _():
            h_ref[pl.ds(i * _BLK, _BLK), :] = jnp.maximum(
                jnp.dot(adj_vmem[pl.ds(i * _BLK, _BLK), :], s2,
                        preferred_element_type=jnp.float32) + bm, 0.0)

        @pl.when(i >= _KB)
        def _():
            am = nz0_ref[...] * adj0_ref[...] + nz1_ref[...] * adj1_ref[...]
            h_ref[pl.ds(i * _BLK, _BLK), :] = jnp.maximum(
                jnp.dot(am, s2, preferred_element_type=jnp.float32) + bm, 0.0)

    @pl.when(p == 3)
    def _phase3():
        @pl.when(i == 0)
        def _():
            s3_ref[...] = jnp.dot(h_ref[...], w2_ref[...],
                                  preferred_element_type=jnp.float32)

        s3 = s3_ref[...]
        b2 = b2_ref[...]

        def _softmax_rows(zz):
            m = jnp.max(zz, axis=1, keepdims=True)
            e = jnp.exp(zz - m)
            return e / jnp.sum(e, axis=1, keepdims=True)

        @pl.when(i < _KB)
        def _():
            zz = jnp.dot(adj_vmem[pl.ds(i * _BLK, _BLK), :], s3,
                         preferred_element_type=jnp.float32) + b2
            out_ref[...] = _softmax_rows(zz)

        @pl.when(i >= _KB)
        def _():
            am = nz0_ref[...] * adj0_ref[...] + nz1_ref[...] * adj1_ref[...]
            zz = jnp.dot(am, s3, preferred_element_type=jnp.float32) + b2
            out_ref[...] = _softmax_rows(zz)


def kernel(adj0, adj1, x, adj_origin, atten_w, atten_b, gcn1_w, gcn1_b,
           gcn_w, gcn_b, gcn2_w, gcn2_b):
    del adj_origin
    n = adj0.shape[0]
    f = x.shape[1]
    h = gcn1_w.shape[1]
    c = gcn2_w.shape[1]
    nblk = n // _BLK
    last = nblk - 1

    ab = atten_b.reshape(1, 1).astype(jnp.float32)
    b1 = gcn1_b.reshape(1, h)
    bm = gcn_b.reshape(1, h)
    b2 = gcn2_b.reshape(1, c)

    def adj0_map(p, i):
        return (jnp.where((p == 0) | (i >= _KB), i, last), 0)

    def adj1_map(p, i):
        return (jnp.where((p <= 1) | (i >= _KB), i, last), 0)

    def const_map(p, i):
        return (0, 0)

    def out_map(p, i):
        return (jnp.where(p == 3, i, 0), 0)

    return pl.pallas_call(
        _gcn_body,
        grid=(4, nblk),
        in_specs=[
            pl.BlockSpec((_BLK, n), adj0_map),
            pl.BlockSpec((_BLK, n), adj1_map),
            pl.BlockSpec((n, f), const_map),
            pl.BlockSpec((1, n), const_map),
            pl.BlockSpec((1, 1), const_map),
            pl.BlockSpec((f, h), const_map),
            pl.BlockSpec((1, h), const_map),
            pl.BlockSpec((h, h), const_map),
            pl.BlockSpec((1, h), const_map),
            pl.BlockSpec((h, c), const_map),
            pl.BlockSpec((1, c), const_map),
        ],
        out_specs=pl.BlockSpec((_BLK, c), out_map),
        out_shape=jax.ShapeDtypeStruct((n, c), jnp.float32),
        scratch_shapes=[
            pltpu.VMEM((_KB * _BLK, n), jnp.float32),
            pltpu.VMEM((n, h), jnp.float32),
            pltpu.VMEM((n, h), jnp.float32),
            pltpu.VMEM((n, c), jnp.float32),
            pltpu.VMEM((1, n), jnp.float32),
            pltpu.VMEM((1, n), jnp.float32),
            pltpu.VMEM((1, n), jnp.float32),
            pltpu.VMEM((1, n), jnp.float32),
        ],
        compiler_params=pltpu.CompilerParams(
            dimension_semantics=("arbitrary", "arbitrary")),
    )(adj0, adj1, x, atten_w, ab, gcn1_w, b1, gcn_w, bm, gcn2_w, b2)


# speedup vs baseline: 1.1604x; 1.1604x over previous
import jax
import jax.numpy as jnp
from jax import lax
from jax.experimental import pallas as pl
from jax.experimental.pallas import tpu as pltpu

_BLK = 256
_KB = 6


def _gcn_body(adj0_ref, adj1_ref, x_ref, aw_ref, ab_ref, w1_ref, b1_ref,
              wm_ref, bm_ref, w2_ref, b2_ref, out_ref,
              adj_vmem, h_ref, s_ref, s3_ref,
              z1_ref, z2_ref, nz0_ref, nz1_ref):
    p = pl.program_id(0)
    i = pl.program_id(1)

    @pl.when(p == 0)
    def _phase0():
        a0 = adj0_ref[...]
        a1 = adj1_ref[...]
        w = aw_ref[...]
        dn = (((1,), (1,)), ((), ()))
        z1_ref[:, pl.ds(i * _BLK, _BLK)] = lax.dot_general(
            w, a0, dn, preferred_element_type=jnp.float32)
        z2_ref[:, pl.ds(i * _BLK, _BLK)] = lax.dot_general(
            w, a1, dn, preferred_element_type=jnp.float32)

        @pl.when(i < _KB)
        def _():
            adj_vmem[pl.ds(i * _BLK, _BLK), :] = a0

    @pl.when(p == 1)
    def _phase1():
        @pl.when(i == 0)
        def _():
            z1 = z1_ref[...] + ab_ref[...]
            z2 = z2_ref[...] + ab_ref[...]
            m = jnp.maximum(z1, z2)
            e1 = jnp.exp(z1 - m)
            e2 = jnp.exp(z2 - m)
            den = e1 + e2
            nz0_ref[...] = e1 / den
            nz1_ref[...] = e2 / den
            s_ref[...] = jnp.dot(x_ref[...], w1_ref[...],
                                  preferred_element_type=jnp.float32)

        nz0 = nz0_ref[...]
        nz1 = nz1_ref[...]
        a1 = adj1_ref[...]
        s1 = s_ref[...]
        b1 = b1_ref[...]

        @pl.when(i < _KB)
        def _():
            am = nz0 * adj_vmem[pl.ds(i * _BLK, _BLK), :] + nz1 * a1
            adj_vmem[pl.ds(i * _BLK, _BLK), :] = am
            h_ref[pl.ds(i * _BLK, _BLK), :] = jnp.maximum(
                jnp.dot(am, s1, preferred_element_type=jnp.float32) + b1, 0.0)

        @pl.when(i >= _KB)
        def _():
            am = nz0 * adj0_ref[...] + nz1 * a1
            h_ref[pl.ds(i * _BLK, _BLK), :] = jnp.maximum(
                jnp.dot(am, s1, preferred_element_type=jnp.float32) + b1, 0.0)

    @pl.when(p == 2)
    def _phase2():
        @pl.when(i == 0)
        def _():
            s_ref[...] = jnp.dot(h_ref[...], wm_ref[...],
                                  preferred_element_type=jnp.float32)

        s2 = s_ref[...]
        bm = bm_ref[...]

        @pl.when(i < _KB)
        def _():
            h_ref[pl.ds(i * _BLK, _BLK), :] = jnp.maximum(
                jnp.dot(adj_vmem[pl.ds(i * _BLK, _BLK), :], s2,
                        preferred_element_type=jnp.float32) + bm, 0.0)

        @pl.when(i >= _KB)
        def _():
            am = nz0_ref[...] * adj0_ref[...] + nz1_ref[...] * adj1_ref[...]
            h_ref[pl.ds(i * _BLK, _BLK), :] = jnp.maximum(
                jnp.dot(am, s2, preferred_element_type=jnp.float32) + bm, 0.0)

    @pl.when(p == 3)
    def _phase3():
        @pl.when(i == 0)
        def _():
            s3_ref[...] = jnp.dot(h_ref[...], w2_ref[...],
                                  preferred_element_type=jnp.float32)

        s3 = s3_ref[...]
        b2 = b2_ref[...]

        def _softmax_rows(zz):
            m = jnp.max(zz, axis=1, keepdims=True)
            e = jnp.exp(zz - m)
            return e / jnp.sum(e, axis=1, keepdims=True)

        @pl.when(i < _KB)
        def _():
            zz = jnp.dot(adj_vmem[pl.ds(i * _BLK, _BLK), :], s3,
                         preferred_element_type=jnp.float32) + b2
            out_ref[...] = _softmax_rows(zz)

        @pl.when(i >= _KB)
        def _():
            am = nz0_ref[...] * adj0_ref[...] + nz1_ref[...] * adj1_ref[...]
            zz = jnp.dot(am, s3, preferred_element_type=jnp.float32) + b2
            out_ref[...] = _softmax_rows(zz)


def kernel(adj0, adj1, x, adj_origin, atten_w, atten_b, gcn1_w, gcn1_b,
           gcn_w, gcn_b, gcn2_w, gcn2_b):
    del adj_origin
    n = adj0.shape[0]
    f = x.shape[1]
    h = gcn1_w.shape[1]
    c = gcn2_w.shape[1]
    nblk = n // _BLK
    last = nblk - 1

    ab = atten_b.reshape(1, 1).astype(jnp.float32)
    b1 = gcn1_b.reshape(1, h)
    bm = gcn_b.reshape(1, h)
    b2 = gcn2_b.reshape(1, c)

    def adj0_map(p, i):
        return (jnp.where((p == 0) | (i >= _KB), i, last), 0)

    def adj1_map(p, i):
        return (jnp.where((p <= 1) | (i >= _KB), i, last), 0)

    def const_map(p, i):
        return (0, 0)

    def out_map(p, i):
        return (jnp.where(p == 3, i, 0), 0)

    return pl.pallas_call(
        _gcn_body,
        grid=(4, nblk),
        in_specs=[
            pl.BlockSpec((_BLK, n), adj0_map),
            pl.BlockSpec((_BLK, n), adj1_map),
            pl.BlockSpec((n, f), const_map),
            pl.BlockSpec((1, n), const_map),
            pl.BlockSpec((1, 1), const_map),
            pl.BlockSpec((f, h), const_map),
            pl.BlockSpec((1, h), const_map),
            pl.BlockSpec((h, h), const_map),
            pl.BlockSpec((1, h), const_map),
            pl.BlockSpec((h, c), const_map),
            pl.BlockSpec((1, c), const_map),
        ],
        out_specs=pl.BlockSpec((_BLK, c), out_map),
        out_shape=jax.ShapeDtypeStruct((n, c), jnp.float32),
        scratch_shapes=[
            pltpu.VMEM((_KB * _BLK, n), jnp.float32),
            pltpu.VMEM((n, h), jnp.float32),
            pltpu.VMEM((n, h), jnp.float32),
            pltpu.VMEM((n, c), jnp.float32),
            pltpu.VMEM((1, n), jnp.float32),
            pltpu.VMEM((1, n), jnp.float32),
            pltpu.VMEM((1, n), jnp.float32),
            pltpu.VMEM((1, n), jnp.float32),
        ],
        compiler_params=pltpu.CompilerParams(
            dimension_semantics=("arbitrary", "arbitrary")),
    )(adj0, adj1, x, atten_w, ab, gcn1_w, b1, gcn_w, bm, gcn2_w, b2)
